# baseline (speedup 1.0000x reference)
import jax
import jax.numpy as jnp
from jax.experimental import pallas as pl
from jax.experimental.pallas import tpu as pltpu

N_NODES = 10000
NUM_ATOM_TYPE = 119
MASK_RATIO = 0.15
NOISE_VAL = 0.1
EPS = 1e-5

_BE = 2000


def _edge_chain_body(m_ref, wpe2_ref, wdT_ref, bd_ref, g_ref, b_ref,
                     wout_ref, out_ref):
    m = m_ref[...]
    pe = jnp.maximum(jnp.dot(m, wpe2_ref[...],
                             preferred_element_type=jnp.float32), 0.0)
    t = jnp.dot(pe, wdT_ref[...],
                preferred_element_type=jnp.float32) + bd_ref[...]
    t = jax.nn.gelu(t)
    mu = jnp.mean(t, axis=-1, keepdims=True)
    var = jnp.mean((t - mu) * (t - mu), axis=-1, keepdims=True)
    tn = (t - mu) * jax.lax.rsqrt(var + EPS) * g_ref[...] + b_ref[...]
    out_ref[...] = jnp.sum(tn * wout_ref[...], axis=-1, keepdims=True)


def _edge_chain(m, W_pe2, dh_dense_w, dh_dense_b, dh_ln_g, dh_ln_b, dh_out_w):
    E = m.shape[0]
    D = m.shape[1]
    grid = (E // _BE,)
    full = lambda i: (0, 0)
    out = pl.pallas_call(
        _edge_chain_body,
        grid=grid,
        in_specs=[
            pl.BlockSpec((_BE, D), lambda i: (i, 0)),
            pl.BlockSpec((D, D), full),
            pl.BlockSpec((D, D), full),
            pl.BlockSpec((1, D), full),
            pl.BlockSpec((1, D), full),
            pl.BlockSpec((1, D), full),
            pl.BlockSpec((1, D), full),
        ],
        out_specs=pl.BlockSpec((_BE, 1), lambda i: (i, 0)),
        out_shape=jax.ShapeDtypeStruct((E, 1), jnp.float32),
    )(m, W_pe2, dh_dense_w.T, dh_dense_b.reshape(1, D),
      dh_ln_g.reshape(1, D), dh_ln_b.reshape(1, D), dh_out_w.reshape(1, D))
    return out


def kernel(x, edge_index, edge_attr, snorm_n, EigVals, EigVecs, W_embed,
           W_edge, w_pe, W_gnn, W_pe2, mlm_dense_w, mlm_dense_b, mlm_ln_g,
           mlm_ln_b, mlm_weight, mlm_bias, dh_dense_w, dh_dense_b, dh_ln_g,
           dh_ln_b, dh_out_w, dh_out_b):
    N = x.shape[0]
    u = jnp.nan_to_num(EigVecs)
    src = edge_index[0]
    dst = edge_index[1]
    mkey = jax.random.key(42)
    perm = jax.random.permutation(mkey, N)
    num_mask = int(MASK_RATIO * N)
    mask_nodes = perm[:num_mask]
    x_masked = x.at[mask_nodes, 0].set(NUM_ATOM_TYPE)
    noise = NOISE_VAL * jax.random.normal(
        jax.random.fold_in(mkey, 1), (num_mask, u.shape[1]),
        dtype=jnp.float32)
    u_masked = u.at[mask_nodes].add(noise)
    PE = jnp.linalg.norm(u[src] - u[dst], axis=-1)
    PE_noise = jnp.linalg.norm(u_masked[src] - u_masked[dst], axis=-1)

    h0 = W_embed[x_masked[:, 0]]
    m = h0[src] + edge_attr @ W_edge + PE_noise[:, None] * w_pe
    agg = jax.ops.segment_sum(m, dst, num_segments=N)
    enc_rep = jax.nn.relu((h0 + agg * snorm_n) @ W_gnn)

    d = _edge_chain(m, W_pe2, dh_dense_w, dh_dense_b, dh_ln_g, dh_ln_b,
                    dh_out_w) + dh_out_b

    feats = enc_rep[mask_nodes]
    h = jax.nn.gelu(feats @ mlm_dense_w.T + mlm_dense_b)
    mu = jnp.mean(h, axis=-1, keepdims=True)
    var = jnp.var(h, axis=-1, keepdims=True)
    h = (h - mu) / jnp.sqrt(var + EPS) * mlm_ln_g + mlm_ln_b
    pred_node = h @ mlm_weight.T + mlm_bias

    row2 = jnp.concatenate([src, dst]).astype(jnp.int64)
    col2 = jnp.concatenate([dst, src]).astype(jnp.int64)
    keys = row2 * N + col2
    M = keys.shape[0]
    order = jnp.argsort(keys)
    keys_sorted = keys[order]
    is_start = jnp.concatenate(
        [jnp.ones((1,), dtype=bool), keys_sorted[1:] != keys_sorted[:-1]])
    seg_sorted = jnp.cumsum(is_start) - 1
    inv_j = jnp.zeros((M,), seg_sorted.dtype).at[order].set(seg_sorted)
    counts_j = jax.ops.segment_sum(jnp.ones((M,), jnp.float32), inv_j,
                                   num_segments=M)
    counts_safe = jnp.maximum(counts_j, 1.0)
    und_dist = jax.ops.segment_sum(jnp.concatenate([d, d], axis=0), inv_j,
                                   num_segments=M) / counts_safe[:, None]
    reconstruct_dist = und_dist[:, 0]
    und_PE = jax.ops.segment_sum(jnp.concatenate([PE, PE]), inv_j,
                                 num_segments=M) / counts_safe
    uk_arr = jnp.zeros((M,), keys.dtype).at[seg_sorted].set(keys_sorted)
    und_row = uk_arr // N
    node_is_masked = jnp.zeros((N,), dtype=bool).at[mask_nodes].set(True)
    sel_mask = (counts_j > 0) & node_is_masked[und_row]
    diff = reconstruct_dist - und_PE
    ad = jnp.abs(diff)
    huber = jnp.where(ad < 1.0, 0.5 * diff * diff, ad - 0.5)
    pe_loss = jnp.sum(jnp.where(sel_mask, huber, 0.0)) / jnp.sum(sel_mask)

    target = x[mask_nodes, 0]
    logp = jax.nn.log_softmax(pred_node, axis=-1)
    atom_loss = -jnp.mean(logp[jnp.arange(num_mask), target])
    return 1.0 * atom_loss + 1.0 * pe_loss
